# Initial kernel scaffold; baseline (speedup 1.0000x reference)
import jax
import jax.numpy as jnp
from jax.experimental import pallas as pl


def kernel(x, label):
    raise NotImplementedError("write your pallas kernel here")


# speedup vs baseline: 3.9784x; 3.9784x over previous
import math

import jax
import jax.numpy as jnp
from jax.experimental import pallas as pl
from jax.experimental.pallas import tpu as pltpu

N_CLASS = 1000
BATCH = 4096
M = 0.2
S = 30.0
EPS = 0.1

_COS_M = math.cos(M)
_SIN_M = math.sin(M)
_TH = math.cos(M)
_MM = math.sin(math.pi - M) * M

BLK = 512
NBLK = BATCH // BLK


def _aam_per_row(vals, onehot):
    c_lab = jnp.sum(vals * onehot, axis=1, keepdims=True)
    sine = jnp.sqrt(jnp.clip(1.0 - c_lab * c_lab, 0.0, 1.0))
    phi = c_lab * _COS_M - sine * _SIN_M
    phi = jnp.where(c_lab - _TH > 0, phi, c_lab - _MM)
    delta = S * (phi - c_lab)
    t = S * vals
    t_mod = t + onehot * delta
    m = jnp.max(t_mod, axis=1, keepdims=True)
    sumexp = jnp.sum(jnp.exp(t_mod - m), axis=1, keepdims=True)
    lse = m + jnp.log(sumexp)
    rmean = (jnp.sum(t, axis=1, keepdims=True) + delta) / N_CLASS
    nll = lse - S * phi
    smooth = lse - rmean
    return (1.0 - EPS) * nll + EPS * smooth


def _sample_body(x_ref, lab_ref, psum_ref, cnt_ref, sums_ref):
    i = pl.program_id(0)
    x = x_ref[...]
    labs = lab_ref[0]
    cols = jax.lax.broadcasted_iota(jnp.int32, (BLK, N_CLASS), 1)
    onehot = (cols == labs).astype(x.dtype)
    per = _aam_per_row(x, onehot)

    @pl.when(i == 0)
    def _():
        psum_ref[...] = jnp.zeros_like(psum_ref)
        cnt_ref[...] = jnp.zeros_like(cnt_ref)
        sums_ref[...] = jnp.zeros_like(sums_ref)

    psum_ref[...] += jnp.sum(per, axis=0, keepdims=True)
    cnt_ref[...] += jnp.sum(onehot, axis=0, keepdims=True)
    sums_ref[...] += jax.lax.dot_general(
        onehot, x, (((0,), (0,)), ((), ())), preferred_element_type=jnp.float32
    )


def _center_body(sums_ref, cntcol_ref, csum_ref, npres_ref):
    sums = sums_ref[...]
    cnt = cntcol_ref[...]
    present = cnt > 0
    inv = jnp.where(present, 1.0 / jnp.where(present, cnt, 1.0), 0.0)
    centers = sums * inv
    rows = jax.lax.broadcasted_iota(jnp.int32, (N_CLASS, N_CLASS), 0)
    cols = jax.lax.broadcasted_iota(jnp.int32, (N_CLASS, N_CLASS), 1)
    diag = (rows == cols).astype(sums.dtype)
    per = _aam_per_row(centers, diag)
    per = jnp.where(present, per, 0.0)
    csum_ref[...] = jnp.sum(per, axis=0, keepdims=True)
    npres_ref[...] = jnp.sum(present.astype(jnp.float32), axis=0, keepdims=True)


def kernel(x, label):
    lab3 = label.reshape(NBLK, BLK, 1)
    psum, cnt, sums = pl.pallas_call(
        _sample_body,
        grid=(NBLK,),
        in_specs=[
            pl.BlockSpec((BLK, N_CLASS), lambda i: (i, 0)),
            pl.BlockSpec((1, BLK, 1), lambda i: (i, 0, 0)),
        ],
        out_specs=[
            pl.BlockSpec((1, 1), lambda i: (0, 0)),
            pl.BlockSpec((1, N_CLASS), lambda i: (0, 0)),
            pl.BlockSpec((N_CLASS, N_CLASS), lambda i: (0, 0)),
        ],
        out_shape=[
            jax.ShapeDtypeStruct((1, 1), jnp.float32),
            jax.ShapeDtypeStruct((1, N_CLASS), jnp.float32),
            jax.ShapeDtypeStruct((N_CLASS, N_CLASS), jnp.float32),
        ],
    )(x, lab3)

    cntcol = cnt.reshape(N_CLASS, 1)
    csum, npres = pl.pallas_call(
        _center_body,
        out_shape=[
            jax.ShapeDtypeStruct((1, 1), jnp.float32),
            jax.ShapeDtypeStruct((1, 1), jnp.float32),
        ],
    )(sums, cntcol)

    loss = 0.5 * psum[0, 0] / BATCH + 0.5 * csum[0, 0] / npres[0, 0]
    return loss
